# baseline (speedup 1.0000x reference)
import functools

import jax
import jax.numpy as jnp
from jax import lax
from jax.experimental import pallas as pl
from jax.experimental.pallas import tpu as pltpu
from jax.experimental.pallas import tpu_sc as plsc

HIDDEN = 1024
FMIN = 0.1
FMAX = 0.5

B = 128
F = 32768
L = 16
NC = 2
NS = 16
NW = NC * NS
ROWS_PER_W = B // NW
NVEC = F // L

_mesh = plsc.VectorSubcoreMesh(core_axis_name="c", subcore_axis_name="s")

_GATHER_DNUMS = lax.GatherDimensionNumbers(
    offset_dims=(), collapsed_slice_dims=(0,), start_index_map=(0,)
)


def _lane_shuffle(v, idx):
    return lax.gather(
        v,
        idx[:, None],
        _GATHER_DNUMS,
        (1,),
        mode=lax.GatherScatterMode.PROMISE_IN_BOUNDS,
    )


@functools.partial(
    pl.kernel,
    mesh=_mesh,
    out_type=jax.ShapeDtypeStruct((B, HIDDEN), jnp.float32),
    scratch_types=[
        pltpu.VMEM((2, F), jnp.float32),
        pltpu.VMEM((ROWS_PER_W, HIDDEN), jnp.float32),
        pltpu.SemaphoreType.DMA,
        pltpu.SemaphoreType.DMA,
    ],
)
def _psd_peak_kernel(x_hbm, out_hbm, buf, outbuf, sem0, sem1):
    wid = lax.axis_index("s") * NC + lax.axis_index("c")
    base_row = wid * ROWS_PER_W
    sems = (sem0, sem1)

    lane_iota = lax.iota(jnp.int32, L)

    copies = [None, None]
    copies[0] = pltpu.async_copy(x_hbm.at[base_row], buf.at[0], sems[0])

    for r in range(ROWS_PER_W):
        b = r % 2
        nb = (r + 1) % 2
        if r + 1 < ROWS_PER_W:
            copies[nb] = pltpu.async_copy(
                x_hbm.at[base_row + (r + 1)], buf.at[nb], sems[nb]
            )
        copies[b].wait()

        row = buf.at[b]

        def body(i, carry):
            m, mi, c = carry
            v = row[pl.ds(i * L, L)]
            gt = v > m
            m = jnp.where(gt, v, m)
            mi = jnp.where(gt, c, mi)
            c = c + L
            return m, mi, c

        m0 = jnp.full((L,), -jnp.inf, jnp.float32)
        i0 = jnp.zeros((L,), jnp.int32)
        m, mi, _ = lax.fori_loop(0, NVEC, body, (m0, i0, lane_iota))

        for d in (1, 2, 4, 8):
            perm = lane_iota ^ d
            m2 = _lane_shuffle(m, perm)
            mi2 = _lane_shuffle(mi, perm)
            take = (m2 > m) | ((m2 == m) & (mi2 < mi))
            m = jnp.where(take, m2, m)
            mi = jnp.where(take, mi2, mi)

        freq = FMIN + (FMAX - FMIN) * mi.astype(jnp.float32) / (F - 1)
        splat = freq * 60.0
        for j in range(HIDDEN // L):
            outbuf[r, pl.ds(j * L, L)] = splat

    pltpu.sync_copy(outbuf, out_hbm.at[pl.ds(base_row, ROWS_PER_W)])


def kernel(x):
    return _psd_peak_kernel(x)


# speedup vs baseline: 1.7430x; 1.7430x over previous
import functools

import jax
import jax.numpy as jnp
from jax import lax
from jax.experimental import pallas as pl
from jax.experimental.pallas import tpu as pltpu
from jax.experimental.pallas import tpu_sc as plsc

HIDDEN = 1024
FMIN = 0.1
FMAX = 0.5

B = 128
F = 32768
L = 16
NC = 2
NS = 16
NW = NC * NS
ROWS_PER_W = B // NW
NVEC = F // L
U = 8

_mesh = plsc.VectorSubcoreMesh(core_axis_name="c", subcore_axis_name="s")

_GATHER_DNUMS = lax.GatherDimensionNumbers(
    offset_dims=(), collapsed_slice_dims=(0,), start_index_map=(0,)
)


def _lane_shuffle(v, idx):
    return lax.gather(
        v,
        idx[:, None],
        _GATHER_DNUMS,
        (1,),
        mode=lax.GatherScatterMode.PROMISE_IN_BOUNDS,
    )


@functools.partial(
    pl.kernel,
    mesh=_mesh,
    out_type=jax.ShapeDtypeStruct((B, HIDDEN), jnp.float32),
    scratch_types=[
        pltpu.VMEM((2, F), jnp.float32),
        pltpu.VMEM((ROWS_PER_W, HIDDEN), jnp.float32),
        pltpu.SemaphoreType.DMA,
        pltpu.SemaphoreType.DMA,
    ],
)
def _psd_peak_kernel(x_hbm, out_hbm, buf, outbuf, sem0, sem1):
    wid = lax.axis_index("s") * NC + lax.axis_index("c")
    base_row = wid * ROWS_PER_W
    sems = (sem0, sem1)

    lane_iota = lax.iota(jnp.int32, L)

    copies = [None, None]
    copies[0] = pltpu.async_copy(x_hbm.at[base_row], buf.at[0], sems[0])

    for r in range(ROWS_PER_W):
        b = r % 2
        nb = (r + 1) % 2
        if r + 1 < ROWS_PER_W:
            copies[nb] = pltpu.async_copy(
                x_hbm.at[base_row + (r + 1)], buf.at[nb], sems[nb]
            )
        copies[b].wait()

        row = buf.at[b]

        def body(i, carry):
            ms, mis = carry
            new_ms, new_mis = [], []
            for u in range(U):
                j = i * U + u
                v = row[pl.ds(j * L, L)]
                jv = jnp.full((L,), j, jnp.int32)
                gt = v > ms[u]
                new_ms.append(jnp.where(gt, v, ms[u]))
                new_mis.append(jnp.where(gt, jv, mis[u]))
            return tuple(new_ms), tuple(new_mis)

        m0 = tuple(jnp.full((L,), -jnp.inf, jnp.float32) for _ in range(U))
        i0 = tuple(jnp.zeros((L,), jnp.int32) for _ in range(U))
        ms, mis = lax.fori_loop(0, NVEC // U, body, (m0, i0))

        pairs = [(ms[u], mis[u] * L + lane_iota) for u in range(U)]
        while len(pairs) > 1:
            nxt = []
            for p in range(0, len(pairs), 2):
                m1, i1 = pairs[p]
                m2, i2 = pairs[p + 1]
                take = (m2 > m1) | ((m2 == m1) & (i2 < i1))
                nxt.append((jnp.where(take, m2, m1), jnp.where(take, i2, i1)))
            pairs = nxt
        m, mi = pairs[0]

        for d in (1, 2, 4, 8):
            perm = lane_iota ^ d
            m2 = _lane_shuffle(m, perm)
            mi2 = _lane_shuffle(mi, perm)
            take = (m2 > m) | ((m2 == m) & (mi2 < mi))
            m = jnp.where(take, m2, m)
            mi = jnp.where(take, mi2, mi)

        freq = FMIN + (FMAX - FMIN) * mi.astype(jnp.float32) / (F - 1)
        splat = freq * 60.0
        for j in range(HIDDEN // L):
            outbuf[r, pl.ds(j * L, L)] = splat

    pltpu.sync_copy(outbuf, out_hbm.at[pl.ds(base_row, ROWS_PER_W)])


def kernel(x):
    return _psd_peak_kernel(x)


# speedup vs baseline: 3.5361x; 2.0288x over previous
import functools

import jax
import jax.numpy as jnp
from jax.experimental import pallas as pl
from jax.experimental.pallas import tpu as pltpu

HIDDEN = 1024
FMIN = 0.1
FMAX = 0.5

B = 128
F = 32768
BK = 2048
NBLK = F // BK


def _psd_peak_body(x_ref, out_ref, rmax, ridx):
    k = pl.program_id(0)
    blk = x_ref[...]
    bmax = jnp.max(blk, axis=1, keepdims=True)
    iota = jax.lax.broadcasted_iota(jnp.int32, (B, BK), 1)
    cand = jnp.where(blk == bmax, iota, F)
    bidx = jnp.min(cand, axis=1, keepdims=True)

    @pl.when(k == 0)
    def _():
        rmax[...] = bmax
        ridx[...] = bidx

    @pl.when(k > 0)
    def _():
        better = bmax > rmax[...]
        ridx[...] = jnp.where(better, bidx + k * BK, ridx[...])
        rmax[...] = jnp.where(better, bmax, rmax[...])

    @pl.when(k == NBLK - 1)
    def _():
        idxf = ridx[...].astype(jnp.float32)
        freq = FMIN + (FMAX - FMIN) * idxf / (F - 1)
        rr = freq * 60.0
        out_ref[...] = jnp.broadcast_to(rr, (B, HIDDEN))


_psd_peak = pl.pallas_call(
    _psd_peak_body,
    grid=(NBLK,),
    in_specs=[pl.BlockSpec((B, BK), lambda k: (0, k))],
    out_specs=pl.BlockSpec((B, HIDDEN), lambda k: (0, 0)),
    out_shape=jax.ShapeDtypeStruct((B, HIDDEN), jnp.float32),
    scratch_shapes=[
        pltpu.VMEM((B, 1), jnp.float32),
        pltpu.VMEM((B, 1), jnp.int32),
    ],
)


def kernel(x):
    return _psd_peak(x)


# speedup vs baseline: 5.0970x; 1.4414x over previous
import functools

import jax
import jax.numpy as jnp
from jax.experimental import pallas as pl
from jax.experimental.pallas import tpu as pltpu

HIDDEN = 1024
FMIN = 0.1
FMAX = 0.5

B = 128
F = 32768
BK = 4096
NBLK = F // BK


def _psd_peak_body(x_ref, out_ref, rmax, ridx):
    k = pl.program_id(0)
    blk = x_ref[...]
    bmax = jnp.max(blk, axis=1, keepdims=True)
    iota = jax.lax.broadcasted_iota(jnp.int32, (B, BK), 1)
    cand = jnp.where(blk == bmax, iota, F)
    bidx = jnp.min(cand, axis=1, keepdims=True)

    @pl.when(k == 0)
    def _():
        rmax[...] = bmax
        ridx[...] = bidx

    @pl.when(k > 0)
    def _():
        better = bmax > rmax[...]
        ridx[...] = jnp.where(better, bidx + k * BK, ridx[...])
        rmax[...] = jnp.where(better, bmax, rmax[...])

    @pl.when(k == NBLK - 1)
    def _():
        idxf = ridx[...].astype(jnp.float32)
        freq = FMIN + (FMAX - FMIN) * idxf / (F - 1)
        rr = freq * 60.0
        out_ref[...] = jnp.broadcast_to(rr, (B, HIDDEN))


_psd_peak = pl.pallas_call(
    _psd_peak_body,
    grid=(NBLK,),
    in_specs=[pl.BlockSpec((B, BK), lambda k: (0, k))],
    out_specs=pl.BlockSpec((B, HIDDEN), lambda k: (0, 0)),
    out_shape=jax.ShapeDtypeStruct((B, HIDDEN), jnp.float32),
    scratch_shapes=[
        pltpu.VMEM((B, 1), jnp.float32),
        pltpu.VMEM((B, 1), jnp.int32),
    ],
)


def kernel(x):
    return _psd_peak(x)


# speedup vs baseline: 6.3089x; 1.2378x over previous
import functools

import jax
import jax.numpy as jnp
from jax.experimental import pallas as pl
from jax.experimental.pallas import tpu as pltpu

HIDDEN = 1024
FMIN = 0.1
FMAX = 0.5

B = 128
F = 32768
BK = 8192
NBLK = F // BK


def _psd_peak_body(x_ref, out_ref, rmax, ridx):
    k = pl.program_id(0)
    blk = x_ref[...]
    bmax = jnp.max(blk, axis=1, keepdims=True)
    iota = jax.lax.broadcasted_iota(jnp.int32, (B, BK), 1)
    cand = jnp.where(blk == bmax, iota, F)
    bidx = jnp.min(cand, axis=1, keepdims=True)

    @pl.when(k == 0)
    def _():
        rmax[...] = bmax
        ridx[...] = bidx

    @pl.when(k > 0)
    def _():
        better = bmax > rmax[...]
        ridx[...] = jnp.where(better, bidx + k * BK, ridx[...])
        rmax[...] = jnp.where(better, bmax, rmax[...])

    @pl.when(k == NBLK - 1)
    def _():
        idxf = ridx[...].astype(jnp.float32)
        freq = FMIN + (FMAX - FMIN) * idxf / (F - 1)
        rr = freq * 60.0
        out_ref[...] = jnp.broadcast_to(rr, (B, HIDDEN))


_psd_peak = pl.pallas_call(
    _psd_peak_body,
    grid=(NBLK,),
    in_specs=[pl.BlockSpec((B, BK), lambda k: (0, k))],
    out_specs=pl.BlockSpec((B, HIDDEN), lambda k: (0, 0)),
    out_shape=jax.ShapeDtypeStruct((B, HIDDEN), jnp.float32),
    scratch_shapes=[
        pltpu.VMEM((B, 1), jnp.float32),
        pltpu.VMEM((B, 1), jnp.int32),
    ],
)


def kernel(x):
    return _psd_peak(x)


# speedup vs baseline: 6.3472x; 1.0061x over previous
import functools

import jax
import jax.numpy as jnp
from jax.experimental import pallas as pl
from jax.experimental.pallas import tpu as pltpu

HIDDEN = 1024
FMIN = 0.1
FMAX = 0.5

B = 128
F = 32768
BK = 16384
NBLK = F // BK


def _psd_peak_body(x_ref, out_ref, rmax, ridx):
    k = pl.program_id(0)
    blk = x_ref[...]
    bmax = jnp.max(blk, axis=1, keepdims=True)
    iota = jax.lax.broadcasted_iota(jnp.int32, (B, BK), 1)
    cand = jnp.where(blk == bmax, iota, F)
    bidx = jnp.min(cand, axis=1, keepdims=True)

    @pl.when(k == 0)
    def _():
        rmax[...] = bmax
        ridx[...] = bidx

    @pl.when(k > 0)
    def _():
        better = bmax > rmax[...]
        ridx[...] = jnp.where(better, bidx + k * BK, ridx[...])
        rmax[...] = jnp.where(better, bmax, rmax[...])

    @pl.when(k == NBLK - 1)
    def _():
        idxf = ridx[...].astype(jnp.float32)
        freq = FMIN + (FMAX - FMIN) * idxf / (F - 1)
        rr = freq * 60.0
        out_ref[...] = jnp.broadcast_to(rr, (B, HIDDEN))


_psd_peak = pl.pallas_call(
    _psd_peak_body,
    grid=(NBLK,),
    in_specs=[pl.BlockSpec((B, BK), lambda k: (0, k))],
    out_specs=pl.BlockSpec((B, HIDDEN), lambda k: (0, 0)),
    out_shape=jax.ShapeDtypeStruct((B, HIDDEN), jnp.float32),
    scratch_shapes=[
        pltpu.VMEM((B, 1), jnp.float32),
        pltpu.VMEM((B, 1), jnp.int32),
    ],
)


def kernel(x):
    return _psd_peak(x)


# speedup vs baseline: 6.4614x; 1.0180x over previous
import jax
import jax.numpy as jnp
from jax.experimental import pallas as pl
from jax.experimental.pallas import tpu as pltpu

HIDDEN = 1024
FMIN = 0.1
FMAX = 0.5

B = 128
F = 32768
BK = 8192
NSTEP = 2


def _block_argmax(blk, col0):
    bmax = jnp.max(blk, axis=1, keepdims=True)
    iota = jax.lax.broadcasted_iota(jnp.int32, (B, BK), 1)
    cand = jnp.where(blk == bmax, iota, F)
    bidx = jnp.min(cand, axis=1, keepdims=True) + col0
    return bmax, bidx


def _merge(m1, i1, m2, i2):
    take2 = (m2 > m1) | ((m2 == m1) & (i2 < i1))
    return jnp.where(take2, m2, m1), jnp.where(take2, i2, i1)


def _psd_peak_body(xa_ref, xb_ref, out_ref, rmax, ridx):
    k = pl.program_id(0)
    bmax_a, bidx_a = _block_argmax(xa_ref[...], k * BK)
    bmax_b, bidx_b = _block_argmax(xb_ref[...], (k + NSTEP) * BK)
    bmax, bidx = _merge(bmax_a, bidx_a, bmax_b, bidx_b)

    @pl.when(k == 0)
    def _():
        rmax[...] = bmax
        ridx[...] = bidx

    @pl.when(k > 0)
    def _():
        m, i = _merge(rmax[...], ridx[...], bmax, bidx)
        rmax[...] = m
        ridx[...] = i

    @pl.when(k == NSTEP - 1)
    def _():
        idxf = ridx[...].astype(jnp.float32)
        freq = FMIN + (FMAX - FMIN) * idxf / (F - 1)
        rr = freq * 60.0
        out_ref[...] = jnp.broadcast_to(rr, (B, HIDDEN))


_psd_peak = pl.pallas_call(
    _psd_peak_body,
    grid=(NSTEP,),
    in_specs=[
        pl.BlockSpec((B, BK), lambda k: (0, k)),
        pl.BlockSpec((B, BK), lambda k: (0, k + NSTEP)),
    ],
    out_specs=pl.BlockSpec((B, HIDDEN), lambda k: (0, 0)),
    out_shape=jax.ShapeDtypeStruct((B, HIDDEN), jnp.float32),
    scratch_shapes=[
        pltpu.VMEM((B, 1), jnp.float32),
        pltpu.VMEM((B, 1), jnp.int32),
    ],
)


def kernel(x):
    return _psd_peak(x, x)
